# baseline (speedup 1.0000x reference)
import functools

import jax
import jax.numpy as jnp
from jax import lax
from jax.experimental import pallas as pl
from jax.experimental.pallas import tpu as pltpu
from jax.experimental.pallas import tpu_sc as plsc

N_NODES = 10000
N_EDGES = 320000
D_IN = 128
D_HID = 128
D_OUT = 16

NC = 2
NS = 16
NW = NC * NS
EPW = N_EDGES // NW
K = 80
CHUNKS = EPW // K
NBUF = 8
S0 = 632
S_LAST = N_NODES - 15 * S0

_MESH = dict(core_axis_name="c", subcore_axis_name="s")


def _striped_copy(s, src_slc, dst_slc):
    @pl.when(s < 15)
    def _():
        start = pl.multiple_of(s * S0, 8)
        pltpu.sync_copy(src_slc(start, S0), dst_slc(start, S0))

    @pl.when(s == 15)
    def _():
        pltpu.sync_copy(src_slc(15 * S0, S_LAST), dst_slc(15 * S0, S_LAST))


def _make_deg_kernel():
    @functools.partial(
        pl.kernel,
        mesh=plsc.VectorSubcoreMesh(**_MESH),
        out_type=jax.ShapeDtypeStruct((NC, N_NODES, 8), jnp.float32),
        scratch_types=[
            pltpu.VMEM((CHUNKS, K), jnp.int32),
            pltpu.VMEM((K, 8), jnp.float32),
            pltpu.VMEM_SHARED((N_NODES, 8), jnp.float32),
            pltpu.SemaphoreType.DMA,
        ],
        compiler_params=pltpu.CompilerParams(use_tc_tiling_on_sc=False),
    )
    def k(dstg_hbm, zeros_hbm, ones_hbm, out_hbm, di_all, ones_v, acc_sh, sem):
        c = lax.axis_index("c")
        s = lax.axis_index("s")
        wid = c * NS + s
        pltpu.sync_copy(ones_hbm, ones_v)
        pltpu.sync_copy(dstg_hbm.at[wid], di_all)
        _striped_copy(s, lambda b, n: zeros_hbm.at[pl.ds(b, n)],
                      lambda b, n: acc_sh.at[pl.ds(b, n)])
        plsc.subcore_barrier()

        def group(g, carry):
            for b in range(NBUF):
                pltpu.async_copy(ones_v, acc_sh.at[di_all.at[g * NBUF + b]],
                                 sem, add=True)
            for b in range(NBUF):
                pltpu.make_async_copy(zeros_hbm.at[pl.ds(0, K)], ones_v,
                                      sem).wait()
            return carry

        lax.fori_loop(0, CHUNKS // NBUF, group, 0)
        for t in range(CHUNKS % NBUF):
            pltpu.async_copy(ones_v,
                             acc_sh.at[di_all.at[(CHUNKS // NBUF) * NBUF + t]],
                             sem, add=True)
        for t in range(CHUNKS % NBUF):
            pltpu.make_async_copy(zeros_hbm.at[pl.ds(0, K)], ones_v,
                                  sem).wait()
        plsc.subcore_barrier()
        _striped_copy(s, lambda b, n: acc_sh.at[pl.ds(b, n)],
                      lambda b, n: out_hbm.at[c, pl.ds(b, n)])

    return k


def _make_scatter_kernel(depth, nbuf):
    groups = CHUNKS // nbuf
    tail = CHUNKS % nbuf

    @functools.partial(
        pl.kernel,
        mesh=plsc.VectorSubcoreMesh(**_MESH),
        out_type=jax.ShapeDtypeStruct((NC, N_NODES, depth), jnp.float32),
        scratch_types=[
            pltpu.VMEM((CHUNKS, K), jnp.int32),
            pltpu.VMEM((CHUNKS, K), jnp.int32),
            pltpu.VMEM((nbuf, K, depth), jnp.float32),
            pltpu.VMEM_SHARED((N_NODES, depth), jnp.float32),
            pltpu.SemaphoreType.DMA((nbuf,)),
        ],
        compiler_params=pltpu.CompilerParams(use_tc_tiling_on_sc=False),
    )
    def k(rows_hbm, srcg_hbm, dstg_hbm, zeros_hbm, out_hbm,
          si_all, di_all, rows_s, acc_sh, gsem):
        c = lax.axis_index("c")
        s = lax.axis_index("s")
        wid = c * NS + s
        pltpu.sync_copy(srcg_hbm.at[wid], si_all)
        pltpu.sync_copy(dstg_hbm.at[wid], di_all)

        @pl.when(c == 0)
        def _():
            _striped_copy(s, lambda b, n: rows_hbm.at[pl.ds(b, n)],
                          lambda b, n: acc_sh.at[pl.ds(b, n)])

        @pl.when(c == 1)
        def _():
            _striped_copy(s, lambda b, n: zeros_hbm.at[pl.ds(b, n)],
                          lambda b, n: acc_sh.at[pl.ds(b, n)])

        plsc.subcore_barrier()

        for b in range(nbuf):
            pltpu.async_copy(rows_hbm.at[si_all.at[b]], rows_s.at[b],
                             gsem.at[b])

        def group(g, carry):
            for b in range(nbuf):
                j = g * nbuf + b
                pltpu.make_async_copy(rows_hbm.at[pl.ds(0, K)], rows_s.at[b],
                                      gsem.at[b]).wait()
                pltpu.sync_copy(rows_s.at[b], acc_sh.at[di_all.at[j]],
                                add=True)

                @pl.when(j + nbuf < CHUNKS)
                def _():
                    pltpu.async_copy(rows_hbm.at[si_all.at[j + nbuf]],
                                     rows_s.at[b], gsem.at[b])

            return carry

        lax.fori_loop(0, groups, group, 0)
        for t in range(tail):
            j = groups * nbuf + t
            b = j % nbuf
            pltpu.make_async_copy(rows_hbm.at[pl.ds(0, K)], rows_s.at[b],
                                  gsem.at[b]).wait()
            pltpu.sync_copy(rows_s.at[b], acc_sh.at[di_all.at[j]], add=True)
        plsc.subcore_barrier()
        _striped_copy(s, lambda b, n: acc_sh.at[pl.ds(b, n)],
                      lambda b, n: out_hbm.at[c, pl.ds(b, n)])

    return k


_deg_kernel = _make_deg_kernel()
_scatter128 = _make_scatter_kernel(D_HID, 3)
_scatter16 = _make_scatter_kernel(D_OUT, 16)

_NB = 10000


def _d_block(degp_ref):
    deg = degp_ref[0, :, 0:1] + degp_ref[1, :, 0:1] + 1.0
    return lax.rsqrt(deg)


def _tc1(x, W1, degp):
    def body(x_ref, w_ref, degp_ref, out_ref):
        out_ref[...] = jnp.dot(x_ref[...], w_ref[...],
                               preferred_element_type=jnp.float32) * _d_block(degp_ref)

    return pl.pallas_call(
        body,
        grid=(N_NODES // _NB,),
        in_specs=[
            pl.BlockSpec((_NB, D_IN), lambda i: (i, 0)),
            pl.BlockSpec((D_IN, D_HID), lambda i: (0, 0)),
            pl.BlockSpec((NC, _NB, 8), lambda i: (0, i, 0)),
        ],
        out_specs=pl.BlockSpec((_NB, D_HID), lambda i: (i, 0)),
        out_shape=jax.ShapeDtypeStruct((N_NODES, D_HID), jnp.float32),
    )(x, W1, degp)


def _tc2(p, degp, b1, W2):
    def body(p_ref, degp_ref, b1_ref, w_ref, out_ref):
        d = _d_block(degp_ref)
        z = (p_ref[0] + p_ref[1]) * d + b1_ref[...]
        z = jnp.maximum(z, 0.0)
        out_ref[...] = jnp.dot(z, w_ref[...],
                               preferred_element_type=jnp.float32) * d

    return pl.pallas_call(
        body,
        grid=(N_NODES // _NB,),
        in_specs=[
            pl.BlockSpec((NC, _NB, D_HID), lambda i: (0, i, 0)),
            pl.BlockSpec((NC, _NB, 8), lambda i: (0, i, 0)),
            pl.BlockSpec((1, D_HID), lambda i: (0, 0)),
            pl.BlockSpec((D_HID, D_OUT), lambda i: (0, 0)),
        ],
        out_specs=pl.BlockSpec((_NB, D_OUT), lambda i: (i, 0)),
        out_shape=jax.ShapeDtypeStruct((N_NODES, D_OUT), jnp.float32),
    )(p, degp, b1, W2)


def _tc3(q, degp, b2):
    def body(q_ref, degp_ref, b2_ref, out_ref):
        d = _d_block(degp_ref)
        out_ref[...] = (q_ref[0] + q_ref[1]) * d + b2_ref[...]

    return pl.pallas_call(
        body,
        grid=(N_NODES // _NB,),
        in_specs=[
            pl.BlockSpec((NC, _NB, D_OUT), lambda i: (0, i, 0)),
            pl.BlockSpec((NC, _NB, 8), lambda i: (0, i, 0)),
            pl.BlockSpec((1, D_OUT), lambda i: (0, 0)),
        ],
        out_specs=pl.BlockSpec((_NB, D_OUT), lambda i: (i, 0)),
        out_shape=jax.ShapeDtypeStruct((N_NODES, D_OUT), jnp.float32),
    )(q, degp, b2)


def kernel(x, edge_index, W1, b1, W2, b2):
    ei = edge_index.astype(jnp.int32)
    srcg = ei[0].reshape(NW, CHUNKS, K)
    dstg = ei[1].reshape(NW, CHUNKS, K)
    zeros16 = jnp.zeros((N_NODES, 16), jnp.float32)
    zeros8 = jnp.zeros((N_NODES, 8), jnp.float32)
    zeros128 = jnp.zeros((N_NODES, D_HID), jnp.float32)
    ones8 = jnp.ones((K, 8), jnp.float32)

    degp = _deg_kernel(dstg, zeros8, ones8)
    hp1 = _tc1(x, W1, degp)
    p = _scatter128(hp1, srcg, dstg, zeros128)
    hp2 = _tc2(p, degp, b1.reshape(1, D_HID), W2)
    q = _scatter16(hp2, srcg, dstg, zeros16)
    return _tc3(q, degp, b2.reshape(1, D_OUT))


# speedup vs baseline: 1.0163x; 1.0163x over previous
import functools

import jax
import jax.numpy as jnp
from jax import lax
from jax.experimental import pallas as pl
from jax.experimental.pallas import tpu as pltpu
from jax.experimental.pallas import tpu_sc as plsc

N_NODES = 10000
N_EDGES = 320000
D_IN = 128
D_HID = 128
D_OUT = 16

NC = 2
NS = 16
NW = NC * NS
EPW = N_EDGES // NW
K = 80
CHUNKS = EPW // K
NBUF = 8
S0 = 632
S_LAST = N_NODES - 15 * S0

_MESH = dict(core_axis_name="c", subcore_axis_name="s")


def _striped_copy(s, src_slc, dst_slc):
    @pl.when(s < 15)
    def _():
        start = pl.multiple_of(s * S0, 8)
        pltpu.sync_copy(src_slc(start, S0), dst_slc(start, S0))

    @pl.when(s == 15)
    def _():
        pltpu.sync_copy(src_slc(15 * S0, S_LAST), dst_slc(15 * S0, S_LAST))


def _make_deg_kernel(k_sz=K):
    chunks = EPW // k_sz

    @functools.partial(
        pl.kernel,
        mesh=plsc.VectorSubcoreMesh(**_MESH),
        out_type=jax.ShapeDtypeStruct((NC, N_NODES, 8), jnp.float32),
        scratch_types=[
            pltpu.VMEM((chunks, k_sz), jnp.int32),
            pltpu.VMEM((k_sz, 8), jnp.float32),
            pltpu.VMEM_SHARED((N_NODES, 8), jnp.float32),
            pltpu.SemaphoreType.DMA,
        ],
        compiler_params=pltpu.CompilerParams(use_tc_tiling_on_sc=False),
    )
    def k(dstg_hbm, zeros_hbm, ones_hbm, out_hbm, di_all, ones_v, acc_sh, sem):
        c = lax.axis_index("c")
        s = lax.axis_index("s")
        wid = c * NS + s
        pltpu.sync_copy(ones_hbm, ones_v)
        pltpu.sync_copy(dstg_hbm.at[wid], di_all)
        _striped_copy(s, lambda b, n: zeros_hbm.at[pl.ds(b, n)],
                      lambda b, n: acc_sh.at[pl.ds(b, n)])
        plsc.subcore_barrier()

        def group(g, carry):
            for b in range(NBUF):
                pltpu.async_copy(ones_v, acc_sh.at[di_all.at[g * NBUF + b]],
                                 sem, add=True)
            for b in range(NBUF):
                pltpu.make_async_copy(zeros_hbm.at[pl.ds(0, k_sz)], ones_v,
                                      sem).wait()
            return carry

        lax.fori_loop(0, chunks // NBUF, group, 0)
        for t in range(chunks % NBUF):
            pltpu.async_copy(ones_v,
                             acc_sh.at[di_all.at[(chunks // NBUF) * NBUF + t]],
                             sem, add=True)
        for t in range(chunks % NBUF):
            pltpu.make_async_copy(zeros_hbm.at[pl.ds(0, k_sz)], ones_v,
                                  sem).wait()
        plsc.subcore_barrier()
        _striped_copy(s, lambda b, n: acc_sh.at[pl.ds(b, n)],
                      lambda b, n: out_hbm.at[c, pl.ds(b, n)])

    return k


def _make_scatter_kernel(depth, nbuf, k_sz=K):
    chunks = EPW // k_sz
    groups = chunks // nbuf
    tail = chunks % nbuf

    @functools.partial(
        pl.kernel,
        mesh=plsc.VectorSubcoreMesh(**_MESH),
        out_type=jax.ShapeDtypeStruct((NC, N_NODES, depth), jnp.float32),
        scratch_types=[
            pltpu.VMEM((chunks, k_sz), jnp.int32),
            pltpu.VMEM((chunks, k_sz), jnp.int32),
            pltpu.VMEM((nbuf, k_sz, depth), jnp.float32),
            pltpu.VMEM_SHARED((N_NODES, depth), jnp.float32),
            pltpu.SemaphoreType.DMA((nbuf,)),
        ],
        compiler_params=pltpu.CompilerParams(use_tc_tiling_on_sc=False),
    )
    def k(rows_hbm, srcg_hbm, dstg_hbm, zeros_hbm, out_hbm,
          si_all, di_all, rows_s, acc_sh, gsem):
        c = lax.axis_index("c")
        s = lax.axis_index("s")
        wid = c * NS + s
        pltpu.sync_copy(srcg_hbm.at[wid], si_all)
        pltpu.sync_copy(dstg_hbm.at[wid], di_all)

        @pl.when(c == 0)
        def _():
            _striped_copy(s, lambda b, n: rows_hbm.at[pl.ds(b, n)],
                          lambda b, n: acc_sh.at[pl.ds(b, n)])

        @pl.when(c == 1)
        def _():
            _striped_copy(s, lambda b, n: zeros_hbm.at[pl.ds(b, n)],
                          lambda b, n: acc_sh.at[pl.ds(b, n)])

        plsc.subcore_barrier()

        for b in range(nbuf):
            pltpu.async_copy(rows_hbm.at[si_all.at[b]], rows_s.at[b],
                             gsem.at[b])

        def group(g, carry):
            for b in range(nbuf):
                j = g * nbuf + b
                pltpu.make_async_copy(rows_hbm.at[pl.ds(0, k_sz)],
                                      rows_s.at[b], gsem.at[b]).wait()
                pltpu.sync_copy(rows_s.at[b], acc_sh.at[di_all.at[j]],
                                add=True)

                @pl.when(j + nbuf < chunks)
                def _():
                    pltpu.async_copy(rows_hbm.at[si_all.at[j + nbuf]],
                                     rows_s.at[b], gsem.at[b])

            return carry

        lax.fori_loop(0, groups, group, 0)
        for t in range(tail):
            j = groups * nbuf + t
            b = j % nbuf
            pltpu.make_async_copy(rows_hbm.at[pl.ds(0, k_sz)], rows_s.at[b],
                                  gsem.at[b]).wait()
            pltpu.sync_copy(rows_s.at[b], acc_sh.at[di_all.at[j]], add=True)
        plsc.subcore_barrier()
        _striped_copy(s, lambda b, n: acc_sh.at[pl.ds(b, n)],
                      lambda b, n: out_hbm.at[c, pl.ds(b, n)])

    return k


_deg_kernel = _make_deg_kernel(125)
_scatter128 = _make_scatter_kernel(D_HID, 3)
_scatter16 = _make_scatter_kernel(D_OUT, 8, 125)

_NB = 5000


def _d_block(degp_ref):
    deg = degp_ref[0, :, 0:1] + degp_ref[1, :, 0:1] + 1.0
    return lax.rsqrt(deg)


def _tc1(x, W1, degp):
    def body(x_ref, w_ref, degp_ref, out_ref):
        out_ref[...] = jnp.dot(x_ref[...], w_ref[...],
                               preferred_element_type=jnp.float32) * _d_block(degp_ref)

    return pl.pallas_call(
        body,
        grid=(N_NODES // _NB,),
        in_specs=[
            pl.BlockSpec((_NB, D_IN), lambda i: (i, 0)),
            pl.BlockSpec((D_IN, D_HID), lambda i: (0, 0)),
            pl.BlockSpec((NC, _NB, 8), lambda i: (0, i, 0)),
        ],
        out_specs=pl.BlockSpec((_NB, D_HID), lambda i: (i, 0)),
        out_shape=jax.ShapeDtypeStruct((N_NODES, D_HID), jnp.float32),
    )(x, W1, degp)


def _tc2(p, degp, b1, W2):
    def body(p_ref, degp_ref, b1_ref, w_ref, out_ref):
        d = _d_block(degp_ref)
        z = (p_ref[0] + p_ref[1]) * d + b1_ref[...]
        z = jnp.maximum(z, 0.0)
        out_ref[...] = jnp.dot(z, w_ref[...],
                               preferred_element_type=jnp.float32) * d

    return pl.pallas_call(
        body,
        grid=(N_NODES // _NB,),
        in_specs=[
            pl.BlockSpec((NC, _NB, D_HID), lambda i: (0, i, 0)),
            pl.BlockSpec((NC, _NB, 8), lambda i: (0, i, 0)),
            pl.BlockSpec((1, D_HID), lambda i: (0, 0)),
            pl.BlockSpec((D_HID, D_OUT), lambda i: (0, 0)),
        ],
        out_specs=pl.BlockSpec((_NB, D_OUT), lambda i: (i, 0)),
        out_shape=jax.ShapeDtypeStruct((N_NODES, D_OUT), jnp.float32),
    )(p, degp, b1, W2)


def _tc3(q, degp, b2):
    def body(q_ref, degp_ref, b2_ref, out_ref):
        d = _d_block(degp_ref)
        out_ref[...] = (q_ref[0] + q_ref[1]) * d + b2_ref[...]

    return pl.pallas_call(
        body,
        grid=(N_NODES // _NB,),
        in_specs=[
            pl.BlockSpec((NC, _NB, D_OUT), lambda i: (0, i, 0)),
            pl.BlockSpec((NC, _NB, 8), lambda i: (0, i, 0)),
            pl.BlockSpec((1, D_OUT), lambda i: (0, 0)),
        ],
        out_specs=pl.BlockSpec((_NB, D_OUT), lambda i: (i, 0)),
        out_shape=jax.ShapeDtypeStruct((N_NODES, D_OUT), jnp.float32),
    )(q, degp, b2)


def kernel(x, edge_index, W1, b1, W2, b2):
    ei = edge_index.astype(jnp.int32)
    srcg = ei[0].reshape(NW, CHUNKS, K)
    dstg = ei[1].reshape(NW, CHUNKS, K)
    srcg125 = ei[0].reshape(NW, EPW // 125, 125)
    dstg125 = ei[1].reshape(NW, EPW // 125, 125)
    zeros16 = jnp.zeros((N_NODES, 16), jnp.float32)
    zeros8 = jnp.zeros((N_NODES, 8), jnp.float32)
    zeros128 = jnp.zeros((N_NODES, D_HID), jnp.float32)
    ones8 = jnp.ones((125, 8), jnp.float32)

    degp = _deg_kernel(dstg125, zeros8, ones8)
    hp1 = _tc1(x, W1, degp)
    p = _scatter128(hp1, srcg, dstg, zeros128)
    hp2 = _tc2(p, degp, b1.reshape(1, D_HID), W2)
    q = _scatter16(hp2, srcg125, dstg125, zeros16)
    return _tc3(q, degp, b2.reshape(1, D_OUT))
